# baseline (speedup 1.0000x reference)
import functools

import jax
import jax.numpy as jnp
from jax import lax
from jax.experimental import pallas as pl
from jax.experimental.pallas import tpu as pltpu
from jax.experimental.pallas import tpu_sc as plsc

_FEATURE_DIM = 4096
_HIDDEN_DIM = 256
_N_EXPERTS = 64
_TOKENS = 16384
_BT = 1024

_NC = 2
_NS = 16
_NW = _NC * _NS
_TPW = _TOKENS // _NW
_L = 16
_GROUPS = _TPW // _L


def _mlp_body(x_ref, w1_ref, b1_ref, w2_ref, b2_ref, logits_ref):
    h = jnp.dot(x_ref[...], w1_ref[...], preferred_element_type=jnp.float32)
    h = jnp.maximum(h + b1_ref[...], 0.0)
    logits = jnp.dot(h, w2_ref[...], preferred_element_type=jnp.float32)
    logits_ref[...] = (logits + b2_ref[...]).T


_NCH = 2
_TPC = _TPW // _NCH
_GPC = _TPC // _L


_NCHAINS = 8
_EPC = _N_EXPERTS // _NCHAINS


def _route_body(logits_hbm, gates_hbm, idxf_hbm, lg_a, lg_b, gt_v, ix_v,
                sem_a, sem_b):
    wid = lax.axis_index("s") * _NC + lax.axis_index("c")
    base = wid * _TPW

    lane = lax.broadcasted_iota(jnp.int32, (_L,), 0)
    neg_inf = jnp.full((_L,), -jnp.inf, jnp.float32)
    izero = jnp.zeros((_L,), jnp.int32)
    fzero = jnp.zeros((_L,), jnp.float32)

    def zrow(r, c):
        gt_v[r, pl.ds(0, _L)] = fzero
        gt_v[r, pl.ds(_L, _L)] = fzero
        gt_v[r, pl.ds(2 * _L, _L)] = fzero
        gt_v[r, pl.ds(3 * _L, _L)] = fzero
        return c

    def merge(a, b):
        am1, ai1, am2, ai2 = a
        bm1, bi1, bm2, bi2 = b
        a_first = am1 >= bm1
        m1 = jnp.where(a_first, am1, bm1)
        i1 = jnp.where(a_first, ai1, bi1)
        s_a = am2 >= bm1
        s_b = am1 >= bm2
        m2 = jnp.where(a_first, jnp.where(s_a, am2, bm1), jnp.where(s_b, am1, bm2))
        i2 = jnp.where(a_first, jnp.where(s_a, ai2, bi1), jnp.where(s_b, ai1, bi2))
        return m1, i1, m2, i2

    def make_group(lg_v):
        def group(g, c):
            off = g * _L
            t0 = off + lane
            chains = []
            for c4 in range(_NCHAINS):
                m1, i1, m2, i2 = neg_inf, izero, neg_inf, izero
                for k in range(_EPC):
                    e = c4 * _EPC + k
                    e_vec = jnp.full((_L,), e, jnp.int32)
                    v = lg_v[e, pl.ds(off, _L)]
                    gt1 = v > m1
                    gt2 = v > m2
                    i2 = jnp.where(gt1, i1, jnp.where(gt2, e_vec, i2))
                    m2 = jnp.where(gt1, m1, jnp.maximum(m2, v))
                    i1 = jnp.where(gt1, e_vec, i1)
                    m1 = jnp.maximum(m1, v)
                chains.append((m1, i1, m2, i2))
            while len(chains) > 1:
                chains = [merge(chains[j], chains[j + 1])
                          for j in range(0, len(chains), 2)]
            m1, i1, m2, i2 = chains[0]
            g1 = 1.0 / (1.0 + jnp.exp(m2 - m1))
            g2 = 1.0 - g1
            plsc.store_scatter(gt_v, [t0, i1], g1)
            plsc.store_scatter(gt_v, [t0, i2], g2)
            t2 = t0 * 2
            plsc.store_scatter(ix_v, [t2], i1)
            plsc.store_scatter(ix_v, [t2 + 1], i2)
            return c

        def pair(p, c):
            group(2 * p, c)
            group(2 * p + 1, c)
            return c
        return pair

    def unzero(g, c):
        t0 = g * _L + lane
        t2 = t0 * 2
        i1 = plsc.load_gather(ix_v, [t2])
        i2 = plsc.load_gather(ix_v, [t2 + 1])
        plsc.store_scatter(gt_v, [t0, i1], fzero)
        plsc.store_scatter(gt_v, [t0, i2], fzero)
        return c

    h_a = pltpu.async_copy(logits_hbm.at[:, pl.ds(base, _TPC)], lg_a, sem_a)
    h_b = pltpu.async_copy(logits_hbm.at[:, pl.ds(base + _TPC, _TPC)], lg_b, sem_b)
    lax.fori_loop(0, _TPC, zrow, 0)
    h_a.wait()
    lax.fori_loop(0, _GPC // 2, make_group(lg_a), 0)
    pltpu.sync_copy(gt_v, gates_hbm.at[pl.ds(base, _TPC)])
    pltpu.sync_copy(ix_v, idxf_hbm.at[pl.ds(base * 2, _TPC * 2)])
    lax.fori_loop(0, _GPC, unzero, 0)
    h_b.wait()
    lax.fori_loop(0, _GPC // 2, make_group(lg_b), 0)
    pltpu.sync_copy(gt_v, gates_hbm.at[pl.ds(base + _TPC, _TPC)])
    pltpu.sync_copy(ix_v, idxf_hbm.at[pl.ds((base + _TPC) * 2, _TPC * 2)])


@jax.jit
def kernel(x, W1, b1, W2, b2):
    grid = (_TOKENS // _BT,)
    logits = pl.pallas_call(
        _mlp_body,
        grid=grid,
        in_specs=[
            pl.BlockSpec((_BT, _FEATURE_DIM), lambda i: (i, 0)),
            pl.BlockSpec((_FEATURE_DIM, _HIDDEN_DIM), lambda i: (0, 0)),
            pl.BlockSpec((1, _HIDDEN_DIM), lambda i: (0, 0)),
            pl.BlockSpec((_HIDDEN_DIM, _N_EXPERTS), lambda i: (0, 0)),
            pl.BlockSpec((1, _N_EXPERTS), lambda i: (0, 0)),
        ],
        out_specs=pl.BlockSpec((_N_EXPERTS, _BT), lambda i: (0, i)),
        out_shape=jax.ShapeDtypeStruct((_N_EXPERTS, _TOKENS), jnp.float32),
    )(x, W1, b1.reshape(1, -1), W2, b2.reshape(1, -1))

    route = pl.kernel(
        _route_body,
        out_type=[
            jax.ShapeDtypeStruct((_TOKENS, _N_EXPERTS), jnp.float32),
            jax.ShapeDtypeStruct((_TOKENS * 2,), jnp.int32),
        ],
        mesh=plsc.VectorSubcoreMesh(core_axis_name="c", subcore_axis_name="s"),
        compiler_params=pltpu.CompilerParams(needs_layout_passes=False),
        scratch_types=[
            pltpu.VMEM((_N_EXPERTS, _TPC), jnp.float32),
            pltpu.VMEM((_N_EXPERTS, _TPC), jnp.float32),
            pltpu.VMEM((_TPC, _N_EXPERTS), jnp.float32),
            pltpu.VMEM((_TPC * 2,), jnp.int32),
            pltpu.SemaphoreType.DMA,
            pltpu.SemaphoreType.DMA,
        ],
    )
    gates, idxf = route(logits)
    return (gates, idxf.reshape(_TOKENS, 2))


# speedup vs baseline: 1.0089x; 1.0089x over previous
import functools

import jax
import jax.numpy as jnp
from jax import lax
from jax.experimental import pallas as pl
from jax.experimental.pallas import tpu as pltpu
from jax.experimental.pallas import tpu_sc as plsc

_FEATURE_DIM = 4096
_HIDDEN_DIM = 256
_N_EXPERTS = 64
_TOKENS = 16384
_BT = 1024

_NC = 2
_NS = 16
_NW = _NC * _NS
_TPW = _TOKENS // _NW
_L = 16
_GROUPS = _TPW // _L


def _mlp_body(x_ref, w1_ref, b1_ref, w2_ref, b2_ref, logits_ref):
    h = jnp.dot(x_ref[...], w1_ref[...], preferred_element_type=jnp.float32)
    h = jnp.maximum(h + b1_ref[...], 0.0)
    logits = jnp.dot(h, w2_ref[...], preferred_element_type=jnp.float32)
    logits_ref[...] = (logits + b2_ref[...]).T


_NCH = 2
_TPC = _TPW // _NCH
_GPC = _TPC // _L


_NCHAINS = 4
_EPC = _N_EXPERTS // _NCHAINS


def _route_body(logits_hbm, gates_hbm, idxf_hbm, lg_a, lg_b, gt_v, ix_v,
                sem_a, sem_b):
    wid = lax.axis_index("s") * _NC + lax.axis_index("c")
    base = wid * _TPW

    lane = lax.broadcasted_iota(jnp.int32, (_L,), 0)
    neg_inf = jnp.full((_L,), -jnp.inf, jnp.float32)
    izero = jnp.zeros((_L,), jnp.int32)
    fzero = jnp.zeros((_L,), jnp.float32)

    def zrow(r, c):
        gt_v[r, pl.ds(0, _L)] = fzero
        gt_v[r, pl.ds(_L, _L)] = fzero
        gt_v[r, pl.ds(2 * _L, _L)] = fzero
        gt_v[r, pl.ds(3 * _L, _L)] = fzero
        return c

    def merge(a, b):
        am1, ai1, am2, ai2 = a
        bm1, bi1, bm2, bi2 = b
        a_first = am1 >= bm1
        m1 = jnp.where(a_first, am1, bm1)
        i1 = jnp.where(a_first, ai1, bi1)
        s_a = am2 >= bm1
        s_b = am1 >= bm2
        m2 = jnp.where(a_first, jnp.where(s_a, am2, bm1), jnp.where(s_b, am1, bm2))
        i2 = jnp.where(a_first, jnp.where(s_a, ai2, bi1), jnp.where(s_b, ai1, bi2))
        return m1, i1, m2, i2

    def make_group(lg_v):
        def group(g, c):
            off = g * _L
            t0 = off + lane
            chains = []
            for c4 in range(_NCHAINS):
                m1, i1, m2, i2 = neg_inf, izero, neg_inf, izero
                for k in range(_EPC):
                    e = c4 * _EPC + k
                    e_vec = jnp.full((_L,), e, jnp.int32)
                    v = lg_v[e, pl.ds(off, _L)]
                    gt1 = v > m1
                    gt2 = v > m2
                    i2 = jnp.where(gt1, i1, jnp.where(gt2, e_vec, i2))
                    m2 = jnp.where(gt1, m1, jnp.maximum(m2, v))
                    i1 = jnp.where(gt1, e_vec, i1)
                    m1 = jnp.maximum(m1, v)
                chains.append((m1, i1, m2, i2))
            while len(chains) > 1:
                chains = [merge(chains[j], chains[j + 1])
                          for j in range(0, len(chains), 2)]
            m1, i1, m2, i2 = chains[0]
            g1 = 1.0 / (1.0 + jnp.exp(m2 - m1))
            g2 = 1.0 - g1
            plsc.store_scatter(gt_v, [t0, i1], g1)
            plsc.store_scatter(gt_v, [t0, i2], g2)
            t2 = t0 * 2
            plsc.store_scatter(ix_v, [t2], i1)
            plsc.store_scatter(ix_v, [t2 + 1], i2)
            return c
        return group

    def unzero(g, c):
        t0 = g * _L + lane
        t2 = t0 * 2
        i1 = plsc.load_gather(ix_v, [t2])
        i2 = plsc.load_gather(ix_v, [t2 + 1])
        plsc.store_scatter(gt_v, [t0, i1], fzero)
        plsc.store_scatter(gt_v, [t0, i2], fzero)
        return c

    h_a = pltpu.async_copy(logits_hbm.at[:, pl.ds(base, _TPC)], lg_a, sem_a)
    h_b = pltpu.async_copy(logits_hbm.at[:, pl.ds(base + _TPC, _TPC)], lg_b, sem_b)
    lax.fori_loop(0, _TPC, zrow, 0)
    h_a.wait()
    lax.fori_loop(0, _GPC, make_group(lg_a), 0)
    pltpu.sync_copy(gt_v, gates_hbm.at[pl.ds(base, _TPC)])
    pltpu.sync_copy(ix_v, idxf_hbm.at[pl.ds(base * 2, _TPC * 2)])
    lax.fori_loop(0, _GPC, unzero, 0)
    h_b.wait()
    lax.fori_loop(0, _GPC, make_group(lg_b), 0)
    pltpu.sync_copy(gt_v, gates_hbm.at[pl.ds(base + _TPC, _TPC)])
    pltpu.sync_copy(ix_v, idxf_hbm.at[pl.ds((base + _TPC) * 2, _TPC * 2)])


@jax.jit
def kernel(x, W1, b1, W2, b2):
    grid = (_TOKENS // _BT,)
    logits = pl.pallas_call(
        _mlp_body,
        grid=grid,
        in_specs=[
            pl.BlockSpec((_BT, _FEATURE_DIM), lambda i: (i, 0)),
            pl.BlockSpec((_FEATURE_DIM, _HIDDEN_DIM), lambda i: (0, 0)),
            pl.BlockSpec((1, _HIDDEN_DIM), lambda i: (0, 0)),
            pl.BlockSpec((_HIDDEN_DIM, _N_EXPERTS), lambda i: (0, 0)),
            pl.BlockSpec((1, _N_EXPERTS), lambda i: (0, 0)),
        ],
        out_specs=pl.BlockSpec((_N_EXPERTS, _BT), lambda i: (0, i)),
        out_shape=jax.ShapeDtypeStruct((_N_EXPERTS, _TOKENS), jnp.float32),
    )(x, W1, b1.reshape(1, -1), W2, b2.reshape(1, -1))

    route = pl.kernel(
        _route_body,
        out_type=[
            jax.ShapeDtypeStruct((_TOKENS, _N_EXPERTS), jnp.float32),
            jax.ShapeDtypeStruct((_TOKENS * 2,), jnp.int32),
        ],
        mesh=plsc.VectorSubcoreMesh(core_axis_name="c", subcore_axis_name="s"),
        compiler_params=pltpu.CompilerParams(needs_layout_passes=False),
        scratch_types=[
            pltpu.VMEM((_N_EXPERTS, _TPC), jnp.float32),
            pltpu.VMEM((_N_EXPERTS, _TPC), jnp.float32),
            pltpu.VMEM((_TPC, _N_EXPERTS), jnp.float32),
            pltpu.VMEM((_TPC * 2,), jnp.int32),
            pltpu.SemaphoreType.DMA,
            pltpu.SemaphoreType.DMA,
        ],
    )
    gates, idxf = route(logits)
    return (gates, idxf.reshape(_TOKENS, 2))
